# baseline (speedup 1.0000x reference)
import functools

import jax
import jax.numpy as jnp
from jax.experimental import pallas as pl
from jax.experimental.pallas import tpu as pltpu
from jax.experimental.pallas import tpu_sc as plsc

TILE_M = 1024
GATHER_WINDOW = 128
COMMITMENT = 0.25


def _embed_sq_body(et_ref, esq_ref):
    et = et_ref[...]
    esq_ref[...] = jnp.sum(et * et, axis=0, keepdims=True)


def _argmin_body(x_ref, et_ref, esq_ref, col_ref, idx_ref, mind_ref):
    x = x_ref[...]
    et = et_ref[...]
    m = x.shape[0]
    k = et.shape[1]
    input_sq = jnp.sum(x * x, axis=1, keepdims=True)
    embed_sq = esq_ref[...]
    dot = jax.lax.dot_general(
        x, et, (((1,), (0,)), ((), ())),
        preferred_element_type=jnp.float32)
    t = input_sq + embed_sq
    dist = jnp.float32(-2.0) * dot + t
    mn = jnp.min(dist, axis=1, keepdims=True)
    col = col_ref[...]
    arg = jnp.min(jnp.where(dist == mn, col, jnp.float32(k)),
                  axis=1, keepdims=True)
    idx_ref[0] = arg.astype(jnp.int32)
    mind_ref[0] = mn


def _loss_body(mind_a_ref, mind_b_ref, out_ref):
    s = jnp.sum(mind_a_ref[...]) + jnp.sum(mind_b_ref[...])
    out_ref[...] = s.reshape(1, 1)


def _embed_sq(emb_t):
    d, k = emb_t.shape
    return pl.pallas_call(
        _embed_sq_body,
        out_shape=jax.ShapeDtypeStruct((1, k), jnp.float32),
    )(emb_t)


def _tc_argmin(flat, emb_t, esq, col):
    rows, d = flat.shape
    k = emb_t.shape[1]
    num_tiles = rows // TILE_M
    return pl.pallas_call(
        _argmin_body,
        grid=(num_tiles,),
        in_specs=[
            pl.BlockSpec((TILE_M, d), lambda i: (i, 0)),
            pl.BlockSpec((d, k), lambda i: (0, 0)),
            pl.BlockSpec((1, k), lambda i: (0, 0)),
            pl.BlockSpec((1, k), lambda i: (0, 0)),
        ],
        out_specs=[
            pl.BlockSpec((1, TILE_M, 1), lambda i: (i, 0, 0)),
            pl.BlockSpec((1, TILE_M, 1), lambda i: (i, 0, 0)),
        ],
        out_shape=[
            jax.ShapeDtypeStruct((num_tiles, TILE_M, 1), jnp.int32),
            jax.ShapeDtypeStruct((num_tiles, TILE_M, 1), jnp.float32),
        ],
        compiler_params=pltpu.CompilerParams(
            dimension_semantics=("parallel",)),
    )(flat, emb_t, esq, col)


def _loss_sum(mind3a, mind3b):
    return pl.pallas_call(
        _loss_body,
        out_shape=jax.ShapeDtypeStruct((1, 1), jnp.float32),
    )(mind3a, mind3b)


def _gather_rows(embedding, indices):
    rows = indices.shape[0]
    d = embedding.shape[1]
    idx2 = indices.reshape(1, rows)
    mesh = plsc.VectorSubcoreMesh(
        core_axis_name="core", subcore_axis_name="subcore")

    @functools.partial(
        pl.kernel,
        out_type=jax.ShapeDtypeStruct((rows, d), jnp.float32),
        mesh=mesh)
    def gather_kernel(x_hbm, i_hbm, o_hbm):
        def body(i_vmem, o_vmem):
            pltpu.sync_copy(x_hbm.at[i_vmem.at[0]], o_vmem)

        pltpu.emit_pipeline(
            body,
            grid=(rows // GATHER_WINDOW,),
            in_specs=[pl.BlockSpec((1, GATHER_WINDOW),
                                   index_map=lambda i: (0, i))],
            out_specs=[pl.BlockSpec((GATHER_WINDOW, d),
                                    index_map=lambda i: (i, 0))],
            core_axis_name=("core", "subcore"),
            dimension_semantics=(pltpu.PARALLEL,),
        )(i_hbm, o_hbm)

    return gather_kernel(embedding, idx2)


def kernel(inputs, embedding):
    input_shape = inputs.shape
    d = embedding.shape[1]
    flat = inputs.reshape(-1, d)
    rows = flat.shape[0]
    emb_t = embedding.T

    esq = _embed_sq(emb_t)
    col = jax.lax.iota(jnp.float32, emb_t.shape[1]).reshape(1, -1)

    half = rows // 2
    idx3a, mind3a = _tc_argmin(flat[:half], emb_t, esq, col)
    idx_a = idx3a.reshape(half)
    quant_a = _gather_rows(embedding, idx_a)
    idx3b, mind3b = _tc_argmin(flat[half:], emb_t, esq, col)
    idx_b = idx3b.reshape(half)
    quant_b = _gather_rows(embedding, idx_b)

    s = _loss_sum(mind3a, mind3b)
    m = s[0, 0] / (rows * d)
    loss = m + COMMITMENT * m

    quantized = jnp.concatenate([quant_a, quant_b], axis=0)
    indices = jnp.concatenate([idx_a, idx_b], axis=0)
    return (quantized.reshape(input_shape), loss,
            indices.reshape(input_shape[0], -1))


# speedup vs baseline: 1.1328x; 1.1328x over previous
import functools

import jax
import jax.numpy as jnp
from jax.experimental import pallas as pl
from jax.experimental.pallas import tpu as pltpu
from jax.experimental.pallas import tpu_sc as plsc

TILE_M = 1024
GATHER_WINDOW = 128
COMMITMENT = 0.25


def _embed_sq_body(et_ref, esq_ref):
    et = et_ref[...]
    esq_ref[...] = jnp.sum(et * et, axis=0, keepdims=True)


def _argmin_body(x_ref, et_ref, esq_ref, col_ref, idx_ref, mind_ref):
    x = x_ref[...]
    et = et_ref[...]
    m = x.shape[0]
    k = et.shape[1]
    input_sq = jnp.sum(x * x, axis=1, keepdims=True)
    embed_sq = esq_ref[...]
    dot = jax.lax.dot_general(
        x, et, (((1,), (0,)), ((), ())),
        preferred_element_type=jnp.float32)
    t = input_sq + embed_sq
    dist = jnp.float32(-2.0) * dot + t
    mn = jnp.min(dist, axis=1, keepdims=True)
    col = col_ref[...]
    arg = jnp.min(jnp.where(dist == mn, col, jnp.float32(k)),
                  axis=1, keepdims=True)
    idx_ref[0] = arg.astype(jnp.int32)
    mind_ref[0] = mn


def _loss_body(mind_ref, out_ref):
    out_ref[...] = jnp.sum(mind_ref[...]).reshape(1, 1)


def _embed_sq(emb_t):
    d, k = emb_t.shape
    return pl.pallas_call(
        _embed_sq_body,
        out_shape=jax.ShapeDtypeStruct((1, k), jnp.float32),
    )(emb_t)


def _tc_argmin(flat, emb_t, esq, col):
    rows, d = flat.shape
    k = emb_t.shape[1]
    num_tiles = rows // TILE_M
    return pl.pallas_call(
        _argmin_body,
        grid=(num_tiles,),
        in_specs=[
            pl.BlockSpec((TILE_M, d), lambda i: (i, 0)),
            pl.BlockSpec((d, k), lambda i: (0, 0)),
            pl.BlockSpec((1, k), lambda i: (0, 0)),
            pl.BlockSpec((1, k), lambda i: (0, 0)),
        ],
        out_specs=[
            pl.BlockSpec((1, TILE_M, 1), lambda i: (i, 0, 0)),
            pl.BlockSpec((1, TILE_M, 1), lambda i: (i, 0, 0)),
        ],
        out_shape=[
            jax.ShapeDtypeStruct((num_tiles, TILE_M, 1), jnp.int32),
            jax.ShapeDtypeStruct((num_tiles, TILE_M, 1), jnp.float32),
        ],
        compiler_params=pltpu.CompilerParams(
            dimension_semantics=("parallel",)),
    )(flat, emb_t, esq, col)


def _loss_sum(mind3):
    return pl.pallas_call(
        _loss_body,
        out_shape=jax.ShapeDtypeStruct((1, 1), jnp.float32),
    )(mind3)


def _gather_rows(embedding, indices):
    rows = indices.shape[0]
    d = embedding.shape[1]
    idx2 = indices.reshape(1, rows)
    mesh = plsc.VectorSubcoreMesh(
        core_axis_name="core", subcore_axis_name="subcore")

    @functools.partial(
        pl.kernel,
        out_type=jax.ShapeDtypeStruct((rows, d), jnp.float32),
        mesh=mesh)
    def gather_kernel(x_hbm, i_hbm, o_hbm):
        def body(i_vmem, o_vmem):
            pltpu.sync_copy(x_hbm.at[i_vmem.at[0]], o_vmem)

        pltpu.emit_pipeline(
            body,
            grid=(rows // GATHER_WINDOW,),
            in_specs=[pl.BlockSpec((1, GATHER_WINDOW),
                                   index_map=lambda i: (0, i))],
            out_specs=[pl.BlockSpec((GATHER_WINDOW, d),
                                    index_map=lambda i: (i, 0))],
            core_axis_name=("core", "subcore"),
            dimension_semantics=(pltpu.PARALLEL,),
        )(i_hbm, o_hbm)

    return gather_kernel(embedding, idx2)


def kernel(inputs, embedding):
    input_shape = inputs.shape
    d = embedding.shape[1]
    flat = inputs.reshape(-1, d)
    rows = flat.shape[0]
    emb_t = embedding.T

    esq = _embed_sq(emb_t)
    col = jax.lax.iota(jnp.float32, emb_t.shape[1]).reshape(1, -1)

    idx3, mind3 = _tc_argmin(flat, emb_t, esq, col)
    indices = idx3.reshape(rows)

    s = _loss_sum(mind3)
    m = s[0, 0] / (rows * d)
    loss = m + COMMITMENT * m

    quantized = _gather_rows(embedding, indices)
    return (quantized.reshape(input_shape), loss,
            indices.reshape(input_shape[0], -1))


# speedup vs baseline: 1.1333x; 1.0005x over previous
import functools

import jax
import jax.numpy as jnp
from jax.experimental import pallas as pl
from jax.experimental.pallas import tpu as pltpu
from jax.experimental.pallas import tpu_sc as plsc

TILE_M = 1024
GATHER_WINDOW = 128
COMMITMENT = 0.25


def _embed_sq_body(et_ref, esq_ref):
    et = et_ref[...]
    esq_ref[...] = jnp.sum(et * et, axis=0, keepdims=True)


def _argmin_body(x_ref, et_ref, esq_ref, col_ref, idx_ref, mind_ref):
    x = x_ref[...]
    et = et_ref[...]
    k = et.shape[1]
    input_sq = jnp.sum(x * x, axis=1, keepdims=True)
    embed_sq = esq_ref[...]
    dot = jax.lax.dot_general(
        x, et, (((1,), (0,)), ((), ())),
        preferred_element_type=jnp.float32)
    t = input_sq + embed_sq
    dist = jnp.float32(-2.0) * dot + t
    mn = jnp.min(dist, axis=1, keepdims=True)
    col = col_ref[...]
    arg = jnp.min(jnp.where(dist == mn, col, jnp.float32(k)),
                  axis=1, keepdims=True)
    idx_ref[0] = arg.astype(jnp.int32)
    mind_ref[0] = mn


def _loss_body(mind_ref, out_ref):
    out_ref[...] = jnp.sum(mind_ref[...]).reshape(1, 1)


def _embed_sq(emb_t):
    d, k = emb_t.shape
    return pl.pallas_call(
        _embed_sq_body,
        out_shape=jax.ShapeDtypeStruct((1, k), jnp.float32),
    )(emb_t)


def _tc_argmin(flat, emb_t, esq, col):
    rows, d = flat.shape
    k = emb_t.shape[1]
    num_tiles = rows // TILE_M
    return pl.pallas_call(
        _argmin_body,
        grid=(num_tiles,),
        in_specs=[
            pl.BlockSpec((TILE_M, d), lambda i: (i, 0)),
            pl.BlockSpec((d, k), lambda i: (0, 0)),
            pl.BlockSpec((1, k), lambda i: (0, 0)),
            pl.BlockSpec((1, k), lambda i: (0, 0)),
        ],
        out_specs=[
            pl.BlockSpec((1, TILE_M, 1), lambda i: (i, 0, 0)),
            pl.BlockSpec((1, TILE_M, 1), lambda i: (i, 0, 0)),
        ],
        out_shape=[
            jax.ShapeDtypeStruct((num_tiles, TILE_M, 1), jnp.int32),
            jax.ShapeDtypeStruct((num_tiles, TILE_M, 1), jnp.float32),
        ],
        compiler_params=pltpu.CompilerParams(
            dimension_semantics=("parallel",)),
    )(flat, emb_t, esq, col)


def _loss_sum(mind3):
    return pl.pallas_call(
        _loss_body,
        out_shape=jax.ShapeDtypeStruct((1, 1), jnp.float32),
    )(mind3)


def _gather_rows(embedding, indices):
    rows = indices.shape[0]
    d = embedding.shape[1]
    idx2 = indices.reshape(1, rows)
    mesh = plsc.VectorSubcoreMesh(
        core_axis_name="core", subcore_axis_name="subcore")

    @functools.partial(
        pl.kernel,
        out_type=jax.ShapeDtypeStruct((rows, d), jnp.float32),
        mesh=mesh)
    def gather_kernel(x_hbm, i_hbm, o_hbm):
        def body(i_vmem, o_vmem):
            pltpu.sync_copy(x_hbm.at[i_vmem.at[0]], o_vmem)

        pltpu.emit_pipeline(
            body,
            grid=(rows // GATHER_WINDOW,),
            in_specs=[pl.BlockSpec((1, GATHER_WINDOW),
                                   index_map=lambda i: (0, i))],
            out_specs=[pl.BlockSpec((GATHER_WINDOW, d),
                                    index_map=lambda i: (i, 0))],
            core_axis_name=("core", "subcore"),
            dimension_semantics=(pltpu.PARALLEL,),
        )(i_hbm, o_hbm)

    return gather_kernel(embedding, idx2)


def kernel(inputs, embedding):
    input_shape = inputs.shape
    d = embedding.shape[1]
    flat = inputs.reshape(-1, d)
    rows = flat.shape[0]
    emb_t = embedding.T

    esq = _embed_sq(emb_t)
    col = jax.lax.iota(jnp.float32, emb_t.shape[1]).reshape(1, -1)

    idx3, mind3 = _tc_argmin(flat, emb_t, esq, col)
    indices = idx3.reshape(rows)

    s = _loss_sum(mind3)
    m = s[0, 0] / (rows * d)
    loss = m + COMMITMENT * m

    quantized = _gather_rows(embedding, indices)
    return (quantized.reshape(input_shape), loss,
            indices.reshape(input_shape[0], -1))
